# baseline (speedup 1.0000x reference)
import jax
import jax.numpy as jnp
from jax import lax
from jax.experimental import pallas as pl
from jax.experimental.pallas import tpu as pltpu
from jax.experimental.pallas import tpu_sc as plsc

B = 16384
NB = B // 2
EMB = 32
CTX_DIM = 16
N_CTX = 1000
NC = 2
NS = 16
NW = NC * NS
BPW = NB // NW
G = 8


def _gather_body(user_ids, item_ids, cf0, cf1, cf2, cf3,
                 u_tab, i_tab, c0, c1, c2, c3,
                 u_out, i_out, c0_out, c1_out, c2_out, c3_out,
                 idx_u, idx_i, idx_c0, idx_c1, idx_c2, idx_c3,
                 buf, urows, irows, crows, ctab, sem0, sem1, sem_idx, sem_out):
    wid = lax.axis_index("s") * NC + lax.axis_index("c")
    base = wid * BPW

    idx_cs = (idx_c0, idx_c1, idx_c2, idx_c3)
    cp_u = pltpu.async_copy(
        user_ids.at[pl.ds(base, BPW)], idx_u.at[pl.ds(0, BPW)], sem_idx)
    cp_i = pltpu.async_copy(
        item_ids.at[pl.ds(base, BPW)], idx_i.at[pl.ds(0, BPW)], sem_idx)
    cp_c = [pltpu.async_copy(cf.at[pl.ds(base, BPW)], idx_cs[t], sem_idx)
            for t, cf in enumerate((cf0, cf1, cf2, cf3))]

    iota_lo = lax.iota(jnp.int32, 16)
    iota_hi = iota_lo + 16

    sems = (sem0, sem1)

    def _issue(phase, slot, _tbl, _idxv):
        rvec = _idxv[pl.ds(phase * G, 16)]
        for k in range(G):
            r = rvec[k]
            col0 = pl.multiple_of((r // 128) * 128, 128)
            pltpu.async_copy(
                _tbl.at[:, pl.ds(col0, 128)], buf.at[slot, k], sems[slot])

    def _extract(phase, slot, _tbl, _idxv, _orows):
        rvec = _idxv[pl.ds(phase * G, 16)]
        lvec = lax.rem(rvec, 128)
        for k in range(G):
            pltpu.make_async_copy(
                _tbl.at[:, pl.ds(0, 128)], buf.at[slot, k], sems[slot]).wait()
        for k in range(G):
            lane = jnp.broadcast_to(lvec[k], (16,))
            lo = plsc.load_gather(buf.at[slot, k], [iota_lo, lane])
            hi = plsc.load_gather(buf.at[slot, k], [iota_hi, lane])
            col = jnp.broadcast_to(phase * G + k, (16,))
            plsc.store_scatter(_orows, [iota_lo, col], lo)
            plsc.store_scatter(_orows, [iota_hi, col], hi)

    NPH = BPW // G
    cp_u.wait()
    cp_i.wait()
    for tbl, idxv, orows, rows_out in (
            (u_tab, idx_u, urows, u_out), (i_tab, idx_i, irows, i_out)):
        _issue(0, 0, tbl, idxv)

        @pl.loop(0, NPH // 2)
        def _pipe(it, _tbl=tbl, _idxv=idxv, _orows=orows):
            _issue(2 * it + 1, 1, _tbl, _idxv)
            _extract(2 * it, 0, _tbl, _idxv, _orows)

            @pl.when(it < NPH // 2 - 1)
            def _():
                _issue(2 * it + 2, 0, _tbl, _idxv)

            _extract(2 * it + 1, 1, _tbl, _idxv, _orows)

        pltpu.async_copy(orows, rows_out.at[:, pl.ds(base, BPW)], sem_out)

    for c in cp_c:
        c.wait()
    ctx_tabs = (c0, c1, c2, c3)
    ctx_outs = (c0_out, c1_out, c2_out, c3_out)
    for t in range(4):
        pltpu.sync_copy(ctx_tabs[t], ctab)

        @pl.loop(0, BPW // 16)
        def _cextract(g, _t=t):
            rvec = idx_cs[_t][pl.ds(g * 16, 16)]
            for k in range(16):
                ridx = jnp.broadcast_to(rvec[k], (16,))
                vals = plsc.load_gather(ctab, [iota_lo, ridx])
                col = jnp.broadcast_to(jnp.int32(g * 16 + k), (16,))
                plsc.store_scatter(crows, [iota_lo, col], vals)

        pltpu.sync_copy(crows, ctx_outs[t].at[:, pl.ds(base, BPW)])

    pltpu.make_async_copy(urows, u_out.at[:, pl.ds(base, BPW)], sem_out).wait()
    pltpu.make_async_copy(irows, i_out.at[:, pl.ds(base, BPW)], sem_out).wait()


_gather = pl.kernel(
    _gather_body,
    out_type=(
        jax.ShapeDtypeStruct((EMB, NB), jnp.float32),
        jax.ShapeDtypeStruct((EMB, NB), jnp.float32),
        jax.ShapeDtypeStruct((CTX_DIM, NB), jnp.float32),
        jax.ShapeDtypeStruct((CTX_DIM, NB), jnp.float32),
        jax.ShapeDtypeStruct((CTX_DIM, NB), jnp.float32),
        jax.ShapeDtypeStruct((CTX_DIM, NB), jnp.float32),
    ),
    mesh=plsc.VectorSubcoreMesh(core_axis_name="c", subcore_axis_name="s"),
    compiler_params=pltpu.CompilerParams(needs_layout_passes=False),
    scratch_types=[
        pltpu.VMEM((BPW + 16,), jnp.int32),
        pltpu.VMEM((BPW + 16,), jnp.int32),
        pltpu.VMEM((BPW,), jnp.int32),
        pltpu.VMEM((BPW,), jnp.int32),
        pltpu.VMEM((BPW,), jnp.int32),
        pltpu.VMEM((BPW,), jnp.int32),
        pltpu.VMEM((2, G, EMB, 128), jnp.float32),
        pltpu.VMEM((EMB, BPW), jnp.float32),
        pltpu.VMEM((EMB, BPW), jnp.float32),
        pltpu.VMEM((CTX_DIM, BPW), jnp.float32),
        pltpu.VMEM((CTX_DIM, N_CTX), jnp.float32),
        pltpu.SemaphoreType.DMA,
        pltpu.SemaphoreType.DMA,
        pltpu.SemaphoreType.DMA,
        pltpu.SemaphoreType.DMA,
    ],
)


BS = 2048


def _mlp_body(u, it, c0, c1, c2, c3,
              w1u, w1i, wc0, wc1, wc2, wc3,
              b1, w2, b2, w3, b3, out):
    f32 = jnp.float32
    dn = (((0,), (0,)), ((), ()))
    h = lax.dot_general(u[...], w1u[...], dn, preferred_element_type=f32)
    h += lax.dot_general(it[...], w1i[...], dn, preferred_element_type=f32)
    h += lax.dot_general(c0[...], wc0[...], dn, preferred_element_type=f32)
    h += lax.dot_general(c1[...], wc1[...], dn, preferred_element_type=f32)
    h += lax.dot_general(c2[...], wc2[...], dn, preferred_element_type=f32)
    h += lax.dot_general(c3[...], wc3[...], dn, preferred_element_type=f32)
    h = jnp.maximum(h + b1[...], 0.0)
    h = jnp.maximum(jnp.dot(h, w2[...], preferred_element_type=f32) + b2[...], 0.0)
    out[...] = jnp.dot(h, w3[...], preferred_element_type=f32) + b3[...]


def _full(shape):
    return pl.BlockSpec(shape, lambda i: (0, 0))


_mlp = pl.pallas_call(
    _mlp_body,
    grid=(NB // BS,),
    in_specs=[
        pl.BlockSpec((EMB, BS), lambda i: (0, i)),
        pl.BlockSpec((EMB, BS), lambda i: (0, i)),
        pl.BlockSpec((CTX_DIM, BS), lambda i: (0, i)),
        pl.BlockSpec((CTX_DIM, BS), lambda i: (0, i)),
        pl.BlockSpec((CTX_DIM, BS), lambda i: (0, i)),
        pl.BlockSpec((CTX_DIM, BS), lambda i: (0, i)),
        _full((EMB, 64)),
        _full((EMB, 64)),
        _full((CTX_DIM, 64)),
        _full((CTX_DIM, 64)),
        _full((CTX_DIM, 64)),
        _full((CTX_DIM, 64)),
        _full((1, 64)),
        _full((64, 32)),
        _full((1, 32)),
        _full((32, 1)),
        _full((1, 1)),
    ],
    out_specs=pl.BlockSpec((BS, 1), lambda i: (i, 0)),
    out_shape=jax.ShapeDtypeStruct((NB, 1), jnp.float32),
)


@jax.jit
def kernel(user_ids, item_ids, context_features, user_table, item_table,
           ctx0, ctx1, ctx2, ctx3, W1, b1, W2, b2, W3, b3):
    cf_t = context_features.T
    w1u = W1[:EMB]
    w1i = W1[EMB:2 * EMB]
    wc = [W1[2 * EMB + t * CTX_DIM: 2 * EMB + (t + 1) * CTX_DIM]
          for t in range(4)]
    uT, iT = user_table.T, item_table.T
    cT = (ctx0.T, ctx1.T, ctx2.T, ctx3.T)
    outs = []
    for h in range(2):
        sl = slice(h * NB, (h + 1) * NB)
        feats = _gather(
            user_ids[sl], item_ids[sl],
            cf_t[0, sl], cf_t[1, sl], cf_t[2, sl], cf_t[3, sl],
            uT, iT, *cT)
        outs.append(_mlp(*feats,
                         w1u, w1i, wc[0], wc[1], wc[2], wc[3],
                         b1.reshape(1, 64), W2, b2.reshape(1, 32),
                         W3, b3.reshape(1, 1)))
    return jnp.concatenate(outs, axis=0)


# speedup vs baseline: 1.0462x; 1.0462x over previous
import jax
import jax.numpy as jnp
from jax import lax
from jax.experimental import pallas as pl
from jax.experimental.pallas import tpu as pltpu
from jax.experimental.pallas import tpu_sc as plsc

B = 16384
EMB = 32
CTX_DIM = 16
N_CTX = 1000
NC = 2
NS = 16
NW = NC * NS
BPW = B // NW
G = 8


def _gather_body(user_ids, item_ids, cf0, cf1, cf2, cf3,
                 u_tab, i_tab, c0, c1, c2, c3,
                 u_out, i_out, c0_out, c1_out, c2_out, c3_out,
                 idx_u, idx_i, idx_c0, idx_c1, idx_c2, idx_c3,
                 buf, urows, irows, crows, ctab, sem0, sem1, sem_idx, sem_out):
    wid = lax.axis_index("s") * NC + lax.axis_index("c")
    base = wid * BPW

    idx_cs = (idx_c0, idx_c1, idx_c2, idx_c3)
    cp_u = pltpu.async_copy(
        user_ids.at[pl.ds(base, BPW)], idx_u.at[pl.ds(0, BPW)], sem_idx)
    cp_i = pltpu.async_copy(
        item_ids.at[pl.ds(base, BPW)], idx_i.at[pl.ds(0, BPW)], sem_idx)
    cp_c = [pltpu.async_copy(cf.at[pl.ds(base, BPW)], idx_cs[t], sem_idx)
            for t, cf in enumerate((cf0, cf1, cf2, cf3))]

    iota_lo = lax.iota(jnp.int32, 16)
    iota_hi = iota_lo + 16

    sems = (sem0, sem1)

    def _issue(phase, slot, _tbl, _idxv):
        rvec = _idxv[pl.ds(phase * G, 16)]
        for k in range(G):
            r = rvec[k]
            col0 = pl.multiple_of((r // 128) * 128, 128)
            pltpu.async_copy(
                _tbl.at[:, pl.ds(col0, 128)], buf.at[slot, k], sems[slot])

    def _extract(phase, slot, _tbl, _idxv, _orows):
        rvec = _idxv[pl.ds(phase * G, 16)]
        lvec = lax.rem(rvec, 128)
        for k in range(G):
            pltpu.make_async_copy(
                _tbl.at[:, pl.ds(0, 128)], buf.at[slot, k], sems[slot]).wait()
        for k in range(G):
            lane = jnp.broadcast_to(lvec[k], (16,))
            lo = plsc.load_gather(buf.at[slot, k], [iota_lo, lane])
            hi = plsc.load_gather(buf.at[slot, k], [iota_hi, lane])
            col = jnp.broadcast_to(phase * G + k, (16,))
            plsc.store_scatter(_orows, [iota_lo, col], lo)
            plsc.store_scatter(_orows, [iota_hi, col], hi)

    NPH = BPW // G
    cp_u.wait()
    cp_i.wait()
    for tbl, idxv, orows, rows_out in (
            (u_tab, idx_u, urows, u_out), (i_tab, idx_i, irows, i_out)):
        _issue(0, 0, tbl, idxv)

        @pl.loop(0, NPH // 2)
        def _pipe(it, _tbl=tbl, _idxv=idxv, _orows=orows):
            _issue(2 * it + 1, 1, _tbl, _idxv)
            _extract(2 * it, 0, _tbl, _idxv, _orows)

            @pl.when(it < NPH // 2 - 1)
            def _():
                _issue(2 * it + 2, 0, _tbl, _idxv)

            _extract(2 * it + 1, 1, _tbl, _idxv, _orows)

        pltpu.async_copy(orows, rows_out.at[:, pl.ds(base, BPW)], sem_out)

    for c in cp_c:
        c.wait()
    ctx_tabs = (c0, c1, c2, c3)
    ctx_outs = (c0_out, c1_out, c2_out, c3_out)
    for t in range(4):
        pltpu.sync_copy(ctx_tabs[t], ctab)

        @pl.loop(0, BPW // 16)
        def _cextract(g, _t=t):
            rvec = idx_cs[_t][pl.ds(g * 16, 16)]
            for k in range(16):
                ridx = jnp.broadcast_to(rvec[k], (16,))
                vals = plsc.load_gather(ctab, [iota_lo, ridx])
                col = jnp.broadcast_to(jnp.int32(g * 16 + k), (16,))
                plsc.store_scatter(crows, [iota_lo, col], vals)

        pltpu.sync_copy(crows, ctx_outs[t].at[:, pl.ds(base, BPW)])

    pltpu.make_async_copy(urows, u_out.at[:, pl.ds(base, BPW)], sem_out).wait()
    pltpu.make_async_copy(irows, i_out.at[:, pl.ds(base, BPW)], sem_out).wait()


_gather = pl.kernel(
    _gather_body,
    out_type=(
        jax.ShapeDtypeStruct((EMB, B), jnp.float32),
        jax.ShapeDtypeStruct((EMB, B), jnp.float32),
        jax.ShapeDtypeStruct((CTX_DIM, B), jnp.float32),
        jax.ShapeDtypeStruct((CTX_DIM, B), jnp.float32),
        jax.ShapeDtypeStruct((CTX_DIM, B), jnp.float32),
        jax.ShapeDtypeStruct((CTX_DIM, B), jnp.float32),
    ),
    mesh=plsc.VectorSubcoreMesh(core_axis_name="c", subcore_axis_name="s"),
    compiler_params=pltpu.CompilerParams(needs_layout_passes=False),
    scratch_types=[
        pltpu.VMEM((BPW + 16,), jnp.int32),
        pltpu.VMEM((BPW + 16,), jnp.int32),
        pltpu.VMEM((BPW,), jnp.int32),
        pltpu.VMEM((BPW,), jnp.int32),
        pltpu.VMEM((BPW,), jnp.int32),
        pltpu.VMEM((BPW,), jnp.int32),
        pltpu.VMEM((2, G, EMB, 128), jnp.float32),
        pltpu.VMEM((EMB, BPW), jnp.float32),
        pltpu.VMEM((EMB, BPW), jnp.float32),
        pltpu.VMEM((CTX_DIM, BPW), jnp.float32),
        pltpu.VMEM((CTX_DIM, N_CTX), jnp.float32),
        pltpu.SemaphoreType.DMA,
        pltpu.SemaphoreType.DMA,
        pltpu.SemaphoreType.DMA,
        pltpu.SemaphoreType.DMA,
    ],
)


BS = 2048


def _mlp_body(u, it, c0, c1, c2, c3,
              w1, b1, w2, b2, w3, b3, out):
    f32 = jnp.float32
    dn = (((0,), (0,)), ((), ()))
    feats = jnp.concatenate(
        [u[...], it[...], c0[...], c1[...], c2[...], c3[...]], axis=0)
    h = lax.dot_general(feats, w1[...], dn, preferred_element_type=f32)
    h = jnp.maximum(h + b1[...], 0.0)
    h = jnp.maximum(jnp.dot(h, w2[...], preferred_element_type=f32) + b2[...], 0.0)
    out[...] = jnp.dot(h, w3[...], preferred_element_type=f32) + b3[...]


def _full(shape):
    return pl.BlockSpec(shape, lambda i: (0, 0))


_mlp = pl.pallas_call(
    _mlp_body,
    grid=(B // BS,),
    in_specs=[
        pl.BlockSpec((EMB, BS), lambda i: (0, i)),
        pl.BlockSpec((EMB, BS), lambda i: (0, i)),
        pl.BlockSpec((CTX_DIM, BS), lambda i: (0, i)),
        pl.BlockSpec((CTX_DIM, BS), lambda i: (0, i)),
        pl.BlockSpec((CTX_DIM, BS), lambda i: (0, i)),
        pl.BlockSpec((CTX_DIM, BS), lambda i: (0, i)),
        _full((2 * EMB + 4 * CTX_DIM, 64)),
        _full((1, 64)),
        _full((64, 32)),
        _full((1, 32)),
        _full((32, 1)),
        _full((1, 1)),
    ],
    out_specs=pl.BlockSpec((BS, 1), lambda i: (i, 0)),
    out_shape=jax.ShapeDtypeStruct((B, 1), jnp.float32),
)


@jax.jit
def kernel(user_ids, item_ids, context_features, user_table, item_table,
           ctx0, ctx1, ctx2, ctx3, W1, b1, W2, b2, W3, b3):
    cf_t = context_features.T
    u_e, i_e, c0e, c1e, c2e, c3e = _gather(
        user_ids, item_ids, cf_t[0], cf_t[1], cf_t[2], cf_t[3],
        user_table.T, item_table.T, ctx0.T, ctx1.T, ctx2.T, ctx3.T)
    return _mlp(u_e, i_e, c0e, c1e, c2e, c3e,
                W1, b1.reshape(1, 64), W2, b2.reshape(1, 32),
                W3, b3.reshape(1, 1))


# speedup vs baseline: 1.0551x; 1.0084x over previous
import jax
import jax.numpy as jnp
from jax import lax
from jax.experimental import pallas as pl
from jax.experimental.pallas import tpu as pltpu
from jax.experimental.pallas import tpu_sc as plsc

B = 16384
EMB = 32
CTX_DIM = 16
N_CTX = 1000
NC = 2
NS = 16
NW = NC * NS
BPW = B // NW
G = 8


def _gather_body(user_ids, item_ids, cf0, cf1, cf2, cf3,
                 u_tab, i_tab, c0, c1, c2, c3,
                 u_out, i_out, c0_out, c1_out, c2_out, c3_out,
                 idx_u, idx_i, idx_c0, idx_c1, idx_c2, idx_c3,
                 buf, urows, irows, crows, ctab, sem0, sem1, sem_idx, sem_out):
    wid = lax.axis_index("s") * NC + lax.axis_index("c")
    base = wid * BPW

    idx_cs = (idx_c0, idx_c1, idx_c2, idx_c3)
    cp_u = pltpu.async_copy(
        user_ids.at[pl.ds(base, BPW)], idx_u.at[pl.ds(0, BPW)], sem_idx)
    cp_i = pltpu.async_copy(
        item_ids.at[pl.ds(base, BPW)], idx_i.at[pl.ds(0, BPW)], sem_idx)
    cp_c = [pltpu.async_copy(cf.at[pl.ds(base, BPW)], idx_cs[t], sem_idx)
            for t, cf in enumerate((cf0, cf1, cf2, cf3))]

    iota_lo = lax.iota(jnp.int32, 16)
    iota_hi = iota_lo + 16

    sems = (sem0, sem1)

    def _issue(phase, slot, _tbl, _idxv):
        rvec = _idxv[pl.ds(phase * G, 16)]
        for k in range(G):
            r = rvec[k]
            col0 = pl.multiple_of((r // 128) * 128, 128)
            pltpu.async_copy(
                _tbl.at[:, pl.ds(col0, 128)], buf.at[slot, k], sems[slot])

    def _extract(phase, slot, _tbl, _idxv, _orows):
        rvec = _idxv[pl.ds(phase * G, 16)]
        lvec = lax.rem(rvec, 128)
        for k in range(G):
            pltpu.make_async_copy(
                _tbl.at[:, pl.ds(0, 128)], buf.at[slot, k], sems[slot]).wait()
        for k in range(G):
            lane = jnp.broadcast_to(lvec[k], (16,))
            lo = plsc.load_gather(buf.at[slot, k], [iota_lo, lane])
            hi = plsc.load_gather(buf.at[slot, k], [iota_hi, lane])
            col = jnp.broadcast_to(phase * G + k, (16,))
            plsc.store_scatter(_orows, [iota_lo, col], lo)
            plsc.store_scatter(_orows, [iota_hi, col], hi)

    NPH = BPW // G
    cp_u.wait()
    cp_i.wait()
    for tbl, idxv, orows, rows_out in (
            (u_tab, idx_u, urows, u_out), (i_tab, idx_i, irows, i_out)):
        _issue(0, 0, tbl, idxv)

        @pl.loop(0, NPH // 2)
        def _pipe(it, _tbl=tbl, _idxv=idxv, _orows=orows):
            _issue(2 * it + 1, 1, _tbl, _idxv)
            _extract(2 * it, 0, _tbl, _idxv, _orows)

            @pl.when(it < NPH // 2 - 1)
            def _():
                _issue(2 * it + 2, 0, _tbl, _idxv)

            _extract(2 * it + 1, 1, _tbl, _idxv, _orows)

        pltpu.async_copy(orows, rows_out.at[:, pl.ds(base, BPW)], sem_out)

    for c in cp_c:
        c.wait()
    ctx_tabs = (c0, c1, c2, c3)
    ctx_outs = (c0_out, c1_out, c2_out, c3_out)
    for t in range(4):
        pltpu.sync_copy(ctx_tabs[t], ctab)

        @pl.loop(0, BPW // 16)
        def _cextract(g, _t=t):
            rvec = idx_cs[_t][pl.ds(g * 16, 16)]
            for k in range(16):
                ridx = jnp.broadcast_to(rvec[k], (16,))
                vals = plsc.load_gather(ctab, [iota_lo, ridx])
                col = jnp.broadcast_to(jnp.int32(g * 16 + k), (16,))
                plsc.store_scatter(crows, [iota_lo, col], vals)

        pltpu.sync_copy(crows, ctx_outs[t].at[:, pl.ds(base, BPW)])

    pltpu.make_async_copy(urows, u_out.at[:, pl.ds(base, BPW)], sem_out).wait()
    pltpu.make_async_copy(irows, i_out.at[:, pl.ds(base, BPW)], sem_out).wait()


_gather = pl.kernel(
    _gather_body,
    out_type=(
        jax.ShapeDtypeStruct((EMB, B), jnp.float32),
        jax.ShapeDtypeStruct((EMB, B), jnp.float32),
        jax.ShapeDtypeStruct((CTX_DIM, B), jnp.float32),
        jax.ShapeDtypeStruct((CTX_DIM, B), jnp.float32),
        jax.ShapeDtypeStruct((CTX_DIM, B), jnp.float32),
        jax.ShapeDtypeStruct((CTX_DIM, B), jnp.float32),
    ),
    mesh=plsc.VectorSubcoreMesh(core_axis_name="c", subcore_axis_name="s"),
    compiler_params=pltpu.CompilerParams(needs_layout_passes=False),
    scratch_types=[
        pltpu.VMEM((BPW + 16,), jnp.int32),
        pltpu.VMEM((BPW + 16,), jnp.int32),
        pltpu.VMEM((BPW,), jnp.int32),
        pltpu.VMEM((BPW,), jnp.int32),
        pltpu.VMEM((BPW,), jnp.int32),
        pltpu.VMEM((BPW,), jnp.int32),
        pltpu.VMEM((2, G, EMB, 128), jnp.float32),
        pltpu.VMEM((EMB, BPW), jnp.float32),
        pltpu.VMEM((EMB, BPW), jnp.float32),
        pltpu.VMEM((CTX_DIM, BPW), jnp.float32),
        pltpu.VMEM((CTX_DIM, N_CTX), jnp.float32),
        pltpu.SemaphoreType.DMA,
        pltpu.SemaphoreType.DMA,
        pltpu.SemaphoreType.DMA,
        pltpu.SemaphoreType.DMA,
    ],
)


BS = 4096


def _mlp_body(u, it, c0, c1, c2, c3,
              w1, b1, w2, b2, w3, b3, out):
    f32 = jnp.float32
    dn = (((0,), (0,)), ((), ()))
    feats = jnp.concatenate(
        [u[...], it[...], c0[...], c1[...], c2[...], c3[...]], axis=0)
    h = lax.dot_general(feats, w1[...], dn, preferred_element_type=f32)
    h = jnp.maximum(h + b1[...], 0.0)
    h = jnp.maximum(jnp.dot(h, w2[...], preferred_element_type=f32) + b2[...], 0.0)
    out[...] = jnp.dot(h, w3[...], preferred_element_type=f32) + b3[...]


def _full(shape):
    return pl.BlockSpec(shape, lambda i: (0, 0))


_mlp = pl.pallas_call(
    _mlp_body,
    grid=(B // BS,),
    in_specs=[
        pl.BlockSpec((EMB, BS), lambda i: (0, i)),
        pl.BlockSpec((EMB, BS), lambda i: (0, i)),
        pl.BlockSpec((CTX_DIM, BS), lambda i: (0, i)),
        pl.BlockSpec((CTX_DIM, BS), lambda i: (0, i)),
        pl.BlockSpec((CTX_DIM, BS), lambda i: (0, i)),
        pl.BlockSpec((CTX_DIM, BS), lambda i: (0, i)),
        _full((2 * EMB + 4 * CTX_DIM, 64)),
        _full((1, 64)),
        _full((64, 32)),
        _full((1, 32)),
        _full((32, 1)),
        _full((1, 1)),
    ],
    out_specs=pl.BlockSpec((BS, 1), lambda i: (i, 0)),
    out_shape=jax.ShapeDtypeStruct((B, 1), jnp.float32),
)


@jax.jit
def kernel(user_ids, item_ids, context_features, user_table, item_table,
           ctx0, ctx1, ctx2, ctx3, W1, b1, W2, b2, W3, b3):
    cf_t = context_features.T
    u_e, i_e, c0e, c1e, c2e, c3e = _gather(
        user_ids, item_ids, cf_t[0], cf_t[1], cf_t[2], cf_t[3],
        user_table.T, item_table.T, ctx0.T, ctx1.T, ctx2.T, ctx3.T)
    return _mlp(u_e, i_e, c0e, c1e, c2e, c3e,
                W1, b1.reshape(1, 64), W2, b2.reshape(1, 32),
                W3, b3.reshape(1, 1))


# speedup vs baseline: 1.1417x; 1.0821x over previous
import jax
import jax.numpy as jnp
from jax import lax
from jax.experimental import pallas as pl
from jax.experimental.pallas import tpu as pltpu
from jax.experimental.pallas import tpu_sc as plsc

B = 16384
EMB = 32
CTX_DIM = 16
N_CTX = 1000
NC = 2
NS = 16
NW = NC * NS
BPW = B // NW
G = 4


def _gather_body(user_ids, item_ids, cf0, cf1, cf2, cf3,
                 u_tab, i_tab, c0, c1, c2, c3,
                 u_out, i_out, c0_out, c1_out, c2_out, c3_out,
                 idx_u, idx_i, idx_c0, idx_c1, idx_c2, idx_c3,
                 buf, urows, irows, crows, ctab,
                 sem0, sem1, sem2, sem3, sem_idx, sem_out):
    wid = lax.axis_index("s") * NC + lax.axis_index("c")
    base = wid * BPW

    idx_cs = (idx_c0, idx_c1, idx_c2, idx_c3)
    cp_u = pltpu.async_copy(
        user_ids.at[pl.ds(base, BPW)], idx_u.at[pl.ds(0, BPW)], sem_idx)
    cp_i = pltpu.async_copy(
        item_ids.at[pl.ds(base, BPW)], idx_i.at[pl.ds(0, BPW)], sem_idx)
    cp_c = [pltpu.async_copy(cf.at[pl.ds(base, BPW)], idx_cs[t], sem_idx)
            for t, cf in enumerate((cf0, cf1, cf2, cf3))]

    iota_lo = lax.iota(jnp.int32, 16)
    iota_hi = iota_lo + 16

    sems = (sem0, sem1, sem2, sem3)

    def _issue(phase, slot, _tbl, _idxv):
        rvec = _idxv[pl.ds(phase * G, 16)]
        for k in range(G):
            r = rvec[k]
            col0 = pl.multiple_of((r // 128) * 128, 128)
            pltpu.async_copy(
                _tbl.at[:, pl.ds(col0, 128)], buf.at[slot, k], sems[slot])

    def _extract(phase, slot, _tbl, _idxv, _orows):
        rvec = _idxv[pl.ds(phase * G, 16)]
        lvec = lax.rem(rvec, 128)
        for k in range(G):
            pltpu.make_async_copy(
                _tbl.at[:, pl.ds(0, 128)], buf.at[slot, k], sems[slot]).wait()
        for k in range(G):
            lane = jnp.broadcast_to(lvec[k], (16,))
            lo = plsc.load_gather(buf.at[slot, k], [iota_lo, lane])
            hi = plsc.load_gather(buf.at[slot, k], [iota_hi, lane])
            col = jnp.broadcast_to(phase * G + k, (16,))
            plsc.store_scatter(_orows, [iota_lo, col], lo)
            plsc.store_scatter(_orows, [iota_hi, col], hi)

    NPH = BPW // G
    NIT = NPH // 4
    cp_u.wait()
    cp_i.wait()
    for tbl, idxv, orows, rows_out in (
            (u_tab, idx_u, urows, u_out), (i_tab, idx_i, irows, i_out)):
        _issue(0, 0, tbl, idxv)
        _issue(1, 1, tbl, idxv)
        _issue(2, 2, tbl, idxv)

        @pl.loop(0, NIT)
        def _pipe(it, _tbl=tbl, _idxv=idxv, _orows=orows):
            _issue(4 * it + 3, 3, _tbl, _idxv)
            for s in range(3):
                _extract(4 * it + s, s, _tbl, _idxv, _orows)

                @pl.when(it < NIT - 1)
                def _(_s=s):
                    _issue(4 * it + 4 + _s, _s, _tbl, _idxv)

            _extract(4 * it + 3, 3, _tbl, _idxv, _orows)

        pltpu.async_copy(orows, rows_out.at[:, pl.ds(base, BPW)], sem_out)

    for c in cp_c:
        c.wait()
    ctx_tabs = (c0, c1, c2, c3)
    ctx_outs = (c0_out, c1_out, c2_out, c3_out)
    for t in range(4):
        pltpu.sync_copy(ctx_tabs[t], ctab)

        @pl.loop(0, BPW // 16)
        def _cextract(g, _t=t):
            rvec = idx_cs[_t][pl.ds(g * 16, 16)]
            for k in range(16):
                ridx = jnp.broadcast_to(rvec[k], (16,))
                vals = plsc.load_gather(ctab, [iota_lo, ridx])
                col = jnp.broadcast_to(jnp.int32(g * 16 + k), (16,))
                plsc.store_scatter(crows, [iota_lo, col], vals)

        pltpu.sync_copy(crows, ctx_outs[t].at[:, pl.ds(base, BPW)])

    pltpu.make_async_copy(urows, u_out.at[:, pl.ds(base, BPW)], sem_out).wait()
    pltpu.make_async_copy(irows, i_out.at[:, pl.ds(base, BPW)], sem_out).wait()


_gather = pl.kernel(
    _gather_body,
    out_type=(
        jax.ShapeDtypeStruct((EMB, B), jnp.float32),
        jax.ShapeDtypeStruct((EMB, B), jnp.float32),
        jax.ShapeDtypeStruct((CTX_DIM, B), jnp.float32),
        jax.ShapeDtypeStruct((CTX_DIM, B), jnp.float32),
        jax.ShapeDtypeStruct((CTX_DIM, B), jnp.float32),
        jax.ShapeDtypeStruct((CTX_DIM, B), jnp.float32),
    ),
    mesh=plsc.VectorSubcoreMesh(core_axis_name="c", subcore_axis_name="s"),
    compiler_params=pltpu.CompilerParams(needs_layout_passes=False),
    scratch_types=[
        pltpu.VMEM((BPW + 16,), jnp.int32),
        pltpu.VMEM((BPW + 16,), jnp.int32),
        pltpu.VMEM((BPW,), jnp.int32),
        pltpu.VMEM((BPW,), jnp.int32),
        pltpu.VMEM((BPW,), jnp.int32),
        pltpu.VMEM((BPW,), jnp.int32),
        pltpu.VMEM((4, G, EMB, 128), jnp.float32),
        pltpu.VMEM((EMB, BPW), jnp.float32),
        pltpu.VMEM((EMB, BPW), jnp.float32),
        pltpu.VMEM((CTX_DIM, BPW), jnp.float32),
        pltpu.VMEM((CTX_DIM, N_CTX), jnp.float32),
        pltpu.SemaphoreType.DMA,
        pltpu.SemaphoreType.DMA,
        pltpu.SemaphoreType.DMA,
        pltpu.SemaphoreType.DMA,
        pltpu.SemaphoreType.DMA,
        pltpu.SemaphoreType.DMA,
    ],
)


BS = 4096


def _mlp_body(u, it, c0, c1, c2, c3,
              w1, b1, w2, b2, w3, b3, out):
    f32 = jnp.float32
    dn = (((0,), (0,)), ((), ()))
    feats = jnp.concatenate(
        [u[...], it[...], c0[...], c1[...], c2[...], c3[...]], axis=0)
    h = lax.dot_general(feats, w1[...], dn, preferred_element_type=f32)
    h = jnp.maximum(h + b1[...], 0.0)
    h = jnp.maximum(jnp.dot(h, w2[...], preferred_element_type=f32) + b2[...], 0.0)
    out[...] = jnp.dot(h, w3[...], preferred_element_type=f32) + b3[...]


def _full(shape):
    return pl.BlockSpec(shape, lambda i: (0, 0))


_mlp = pl.pallas_call(
    _mlp_body,
    grid=(B // BS,),
    in_specs=[
        pl.BlockSpec((EMB, BS), lambda i: (0, i)),
        pl.BlockSpec((EMB, BS), lambda i: (0, i)),
        pl.BlockSpec((CTX_DIM, BS), lambda i: (0, i)),
        pl.BlockSpec((CTX_DIM, BS), lambda i: (0, i)),
        pl.BlockSpec((CTX_DIM, BS), lambda i: (0, i)),
        pl.BlockSpec((CTX_DIM, BS), lambda i: (0, i)),
        _full((2 * EMB + 4 * CTX_DIM, 64)),
        _full((1, 64)),
        _full((64, 32)),
        _full((1, 32)),
        _full((32, 1)),
        _full((1, 1)),
    ],
    out_specs=pl.BlockSpec((BS, 1), lambda i: (i, 0)),
    out_shape=jax.ShapeDtypeStruct((B, 1), jnp.float32),
)


@jax.jit
def kernel(user_ids, item_ids, context_features, user_table, item_table,
           ctx0, ctx1, ctx2, ctx3, W1, b1, W2, b2, W3, b3):
    cf_t = context_features.T
    u_e, i_e, c0e, c1e, c2e, c3e = _gather(
        user_ids, item_ids, cf_t[0], cf_t[1], cf_t[2], cf_t[3],
        user_table.T, item_table.T, ctx0.T, ctx1.T, ctx2.T, ctx3.T)
    return _mlp(u_e, i_e, c0e, c1e, c2e, c3e,
                W1, b1.reshape(1, 64), W2, b2.reshape(1, 32),
                W3, b3.reshape(1, 1))


# speedup vs baseline: 1.2031x; 1.0538x over previous
import jax
import jax.numpy as jnp
from jax import lax
from jax.experimental import pallas as pl
from jax.experimental.pallas import tpu as pltpu
from jax.experimental.pallas import tpu_sc as plsc

B = 16384
EMB = 32
CTX_DIM = 16
N_CTX = 1000
NC = 2
NS = 16
NW = NC * NS
BPW = B // NW
G = 2


def _gather_body(user_ids, item_ids, cf0, cf1, cf2, cf3,
                 u_tab, i_tab, c0, c1, c2, c3,
                 u_out, i_out, c0_out, c1_out, c2_out, c3_out,
                 idx_u, idx_i, idx_c0, idx_c1, idx_c2, idx_c3,
                 buf, urows, irows, crows, ctab,
                 sem0, sem1, sem2, sem3, sem4, sem5, sem6, sem7,
                 sem_idx, sem_out):
    wid = lax.axis_index("s") * NC + lax.axis_index("c")
    base = wid * BPW

    idx_cs = (idx_c0, idx_c1, idx_c2, idx_c3)
    cp_u = pltpu.async_copy(
        user_ids.at[pl.ds(base, BPW)], idx_u.at[pl.ds(0, BPW)], sem_idx)
    cp_i = pltpu.async_copy(
        item_ids.at[pl.ds(base, BPW)], idx_i.at[pl.ds(0, BPW)], sem_idx)
    cp_c = [pltpu.async_copy(cf.at[pl.ds(base, BPW)], idx_cs[t], sem_idx)
            for t, cf in enumerate((cf0, cf1, cf2, cf3))]

    iota_lo = lax.iota(jnp.int32, 16)
    iota_hi = iota_lo + 16

    sems = (sem0, sem1, sem2, sem3, sem4, sem5, sem6, sem7)

    def _issue(phase, slot, _tbl, _idxv):
        rvec = _idxv[pl.ds(phase * G, 16)]
        for k in range(G):
            r = rvec[k]
            col0 = pl.multiple_of((r // 128) * 128, 128)
            pltpu.async_copy(
                _tbl.at[:, pl.ds(col0, 128)], buf.at[slot, k], sems[slot])

    def _extract(phase, slot, _tbl, _idxv, _orows):
        rvec = _idxv[pl.ds(phase * G, 16)]
        lvec = lax.rem(rvec, 128)
        for k in range(G):
            pltpu.make_async_copy(
                _tbl.at[:, pl.ds(0, 128)], buf.at[slot, k], sems[slot]).wait()
        for k in range(G):
            lane = jnp.broadcast_to(lvec[k], (16,))
            lo = plsc.load_gather(buf.at[slot, k], [iota_lo, lane])
            hi = plsc.load_gather(buf.at[slot, k], [iota_hi, lane])
            col = jnp.broadcast_to(phase * G + k, (16,))
            plsc.store_scatter(_orows, [iota_lo, col], lo)
            plsc.store_scatter(_orows, [iota_hi, col], hi)

    NPH = BPW // G
    NIT = NPH // 8
    cp_u.wait()
    cp_i.wait()
    for tbl, idxv, orows, rows_out in (
            (u_tab, idx_u, urows, u_out), (i_tab, idx_i, irows, i_out)):
        for s in range(7):
            _issue(s, s, tbl, idxv)

        @pl.loop(0, NIT)
        def _pipe(it, _tbl=tbl, _idxv=idxv, _orows=orows):
            _issue(8 * it + 7, 7, _tbl, _idxv)
            for s in range(7):
                _extract(8 * it + s, s, _tbl, _idxv, _orows)

                @pl.when(it < NIT - 1)
                def _(_s=s):
                    _issue(8 * it + 8 + _s, _s, _tbl, _idxv)

            _extract(8 * it + 7, 7, _tbl, _idxv, _orows)

        pltpu.async_copy(orows, rows_out.at[:, pl.ds(base, BPW)], sem_out)

    for c in cp_c:
        c.wait()
    ctx_tabs = (c0, c1, c2, c3)
    ctx_outs = (c0_out, c1_out, c2_out, c3_out)
    for t in range(4):
        pltpu.sync_copy(ctx_tabs[t], ctab)

        @pl.loop(0, BPW // 16)
        def _cextract(g, _t=t):
            rvec = idx_cs[_t][pl.ds(g * 16, 16)]
            for k in range(16):
                ridx = jnp.broadcast_to(rvec[k], (16,))
                vals = plsc.load_gather(ctab, [iota_lo, ridx])
                col = jnp.broadcast_to(jnp.int32(g * 16 + k), (16,))
                plsc.store_scatter(crows, [iota_lo, col], vals)

        pltpu.sync_copy(crows, ctx_outs[t].at[:, pl.ds(base, BPW)])

    pltpu.make_async_copy(urows, u_out.at[:, pl.ds(base, BPW)], sem_out).wait()
    pltpu.make_async_copy(irows, i_out.at[:, pl.ds(base, BPW)], sem_out).wait()


_gather = pl.kernel(
    _gather_body,
    out_type=(
        jax.ShapeDtypeStruct((EMB, B), jnp.float32),
        jax.ShapeDtypeStruct((EMB, B), jnp.float32),
        jax.ShapeDtypeStruct((CTX_DIM, B), jnp.float32),
        jax.ShapeDtypeStruct((CTX_DIM, B), jnp.float32),
        jax.ShapeDtypeStruct((CTX_DIM, B), jnp.float32),
        jax.ShapeDtypeStruct((CTX_DIM, B), jnp.float32),
    ),
    mesh=plsc.VectorSubcoreMesh(core_axis_name="c", subcore_axis_name="s"),
    compiler_params=pltpu.CompilerParams(needs_layout_passes=False),
    scratch_types=[
        pltpu.VMEM((BPW + 16,), jnp.int32),
        pltpu.VMEM((BPW + 16,), jnp.int32),
        pltpu.VMEM((BPW,), jnp.int32),
        pltpu.VMEM((BPW,), jnp.int32),
        pltpu.VMEM((BPW,), jnp.int32),
        pltpu.VMEM((BPW,), jnp.int32),
        pltpu.VMEM((8, G, EMB, 128), jnp.float32),
        pltpu.VMEM((EMB, BPW), jnp.float32),
        pltpu.VMEM((EMB, BPW), jnp.float32),
        pltpu.VMEM((CTX_DIM, BPW), jnp.float32),
        pltpu.VMEM((CTX_DIM, N_CTX), jnp.float32),
        pltpu.SemaphoreType.DMA,
        pltpu.SemaphoreType.DMA,
        pltpu.SemaphoreType.DMA,
        pltpu.SemaphoreType.DMA,
        pltpu.SemaphoreType.DMA,
        pltpu.SemaphoreType.DMA,
        pltpu.SemaphoreType.DMA,
        pltpu.SemaphoreType.DMA,
        pltpu.SemaphoreType.DMA,
        pltpu.SemaphoreType.DMA,
    ],
)


BS = 4096


def _mlp_body(u, it, c0, c1, c2, c3,
              w1, b1, w2, b2, w3, b3, out):
    f32 = jnp.float32
    dn = (((0,), (0,)), ((), ()))
    feats = jnp.concatenate(
        [u[...], it[...], c0[...], c1[...], c2[...], c3[...]], axis=0)
    h = lax.dot_general(feats, w1[...], dn, preferred_element_type=f32)
    h = jnp.maximum(h + b1[...], 0.0)
    h = jnp.maximum(jnp.dot(h, w2[...], preferred_element_type=f32) + b2[...], 0.0)
    out[...] = jnp.dot(h, w3[...], preferred_element_type=f32) + b3[...]


def _full(shape):
    return pl.BlockSpec(shape, lambda i: (0, 0))


_mlp = pl.pallas_call(
    _mlp_body,
    grid=(B // BS,),
    in_specs=[
        pl.BlockSpec((EMB, BS), lambda i: (0, i)),
        pl.BlockSpec((EMB, BS), lambda i: (0, i)),
        pl.BlockSpec((CTX_DIM, BS), lambda i: (0, i)),
        pl.BlockSpec((CTX_DIM, BS), lambda i: (0, i)),
        pl.BlockSpec((CTX_DIM, BS), lambda i: (0, i)),
        pl.BlockSpec((CTX_DIM, BS), lambda i: (0, i)),
        _full((2 * EMB + 4 * CTX_DIM, 64)),
        _full((1, 64)),
        _full((64, 32)),
        _full((1, 32)),
        _full((32, 1)),
        _full((1, 1)),
    ],
    out_specs=pl.BlockSpec((BS, 1), lambda i: (i, 0)),
    out_shape=jax.ShapeDtypeStruct((B, 1), jnp.float32),
)


@jax.jit
def kernel(user_ids, item_ids, context_features, user_table, item_table,
           ctx0, ctx1, ctx2, ctx3, W1, b1, W2, b2, W3, b3):
    cf_t = context_features.T
    u_e, i_e, c0e, c1e, c2e, c3e = _gather(
        user_ids, item_ids, cf_t[0], cf_t[1], cf_t[2], cf_t[3],
        user_table.T, item_table.T, ctx0.T, ctx1.T, ctx2.T, ctx3.T)
    return _mlp(u_e, i_e, c0e, c1e, c2e, c3e,
                W1, b1.reshape(1, 64), W2, b2.reshape(1, 32),
                W3, b3.reshape(1, 1))
